# baseline (speedup 1.0000x reference)
import functools

import jax
import jax.numpy as jnp
from jax.experimental import pallas as pl
from jax.experimental.pallas import tpu as pltpu


def _dot_nt(a, b):
    return jax.lax.dot_general(
        a, b, (((1,), (1,)), ((), ())),
        preferred_element_type=jnp.float32)


def _rnn_t_kernel(x_ref, wih_ref, whh_ref, b_ref, w1_ref, b1_ref,
                  w2_ref, b2_ref, out_ref, h_ref, *, t_tile, n_chunks):
    ti = pl.program_id(0)

    @pl.when(ti == 0)
    def _():
        h_ref[...] = jnp.zeros_like(h_ref)

    hh, lanes = h_ref.shape
    lc = lanes // n_chunks

    xw = x_ref[:, 0, 0, :]
    pre = _dot_nt(wih_ref[...], xw) + b_ref[...]

    whh_t = whh_ref[...]
    hs = [h_ref[:, c * lc:(c + 1) * lc] for c in range(n_chunks)]
    for t in range(t_tile):
        p = pre[t * hh:(t + 1) * hh, :]
        for c in range(n_chunks):
            mm = jnp.dot(whh_t, hs[c], preferred_element_type=jnp.float32)
            hs[c] = jnp.tanh(p[:, c * lc:(c + 1) * lc] + mm)
    for c in range(n_chunks):
        h_ref[:, c * lc:(c + 1) * lc] = hs[c]

    @pl.when(ti == pl.num_programs(0) - 1)
    def _():
        h = jnp.concatenate(hs, axis=1)
        z1 = jnp.dot(w1_ref[...], h, preferred_element_type=jnp.float32)
        z1 = jnp.maximum(z1 + b1_ref[...], 0.0)
        out = jnp.dot(w2_ref[...], z1, preferred_element_type=jnp.float32)
        out_ref[...] = out + b2_ref[0, 0]


def _pick_t_tile(T, cap=32):
    best = 1
    for cand in range(1, min(T, cap) + 1):
        if T % cand == 0:
            best = cand
    return best


@jax.jit
def kernel(x, w_ih, w_hh, b_ih, b_hh, w_fc1, b_fc1, w_fc2, b_fc2):
    B, T, I = x.shape
    H = w_hh.shape[0]
    F = w_fc1.shape[1]

    t_tile = _pick_t_tile(T)
    nt = T // t_tile
    n_chunks = 2 if B % 256 == 0 else 1

    f32 = jnp.float32
    x3 = x.astype(f32).reshape(B, nt, 1, t_tile * I)

    eye_t = jnp.eye(t_tile, dtype=f32)
    wih_td = jnp.kron(eye_t, w_ih.astype(f32).T)
    whh_t = w_hh.astype(f32).T
    b_rep = jnp.tile((b_ih + b_hh).astype(f32).reshape(H, 1),
                     (t_tile, 1))
    w1_t = w_fc1.astype(f32).T
    b1_col = b_fc1.astype(f32).reshape(F, 1)
    w2_row = w_fc2.astype(f32).reshape(1, F)
    b2 = jnp.asarray(b_fc2, f32).reshape(1, 1)

    cost = pl.CostEstimate(
        flops=2 * T * B * (I * H + H * H) + 2 * B * (H * F + F),
        transcendentals=T * B * H,
        bytes_accessed=4 * (T * B * I + B),
    )

    grid_spec = pltpu.PrefetchScalarGridSpec(
        num_scalar_prefetch=0,
        grid=(nt,),
        in_specs=[
            pl.BlockSpec((B, 1, 1, t_tile * I), lambda t: (0, t, 0, 0)),
            pl.BlockSpec((t_tile * H, t_tile * I), lambda t: (0, 0)),
            pl.BlockSpec((H, H), lambda t: (0, 0)),
            pl.BlockSpec((t_tile * H, 1), lambda t: (0, 0)),
            pl.BlockSpec((F, H), lambda t: (0, 0)),
            pl.BlockSpec((F, 1), lambda t: (0, 0)),
            pl.BlockSpec((1, F), lambda t: (0, 0)),
            pl.BlockSpec((1, 1), lambda t: (0, 0)),
        ],
        out_specs=pl.BlockSpec((1, B), lambda t: (0, 0)),
        scratch_shapes=[pltpu.VMEM((H, B), f32)],
    )

    out = pl.pallas_call(
        functools.partial(_rnn_t_kernel, t_tile=t_tile, n_chunks=n_chunks),
        out_shape=jax.ShapeDtypeStruct((1, B), f32),
        grid_spec=grid_spec,
        compiler_params=pltpu.CompilerParams(
            dimension_semantics=("arbitrary",),
            vmem_limit_bytes=48 * 1024 * 1024,
        ),
        cost_estimate=cost,
    )(x3, wih_td, whh_t, b_rep, w1_t, b1_col, w2_row, b2)

    return out.reshape(B)


# speedup vs baseline: 1.5241x; 1.5241x over previous
import functools

import jax
import jax.numpy as jnp
from jax.experimental import pallas as pl
from jax.experimental.pallas import tpu as pltpu


def _rnn_packed_kernel(x_ref, wih_ref, whh_ref, b_ref, w1_ref, b1_ref,
                       w2_ref, s_ref, b2_ref, out_ref, h_ref,
                       *, t_tile, n_chunks):
    ti = pl.program_id(0)

    @pl.when(ti == 0)
    def _():
        h_ref[...] = jnp.zeros_like(h_ref)

    rows = h_ref.shape[0]
    rc = rows // n_chunks

    pre = (jnp.dot(x_ref[...], wih_ref[...],
                   preferred_element_type=jnp.float32)
           + b_ref[...])

    whh = whh_ref[...]
    hs = [h_ref[c * rc:(c + 1) * rc, :] for c in range(n_chunks)]
    for t in range(t_tile):
        base = t * rows
        for c in range(n_chunks):
            p = pre[base + c * rc:base + (c + 1) * rc, :]
            hs[c] = jnp.tanh(p + jnp.dot(hs[c], whh,
                                         preferred_element_type=jnp.float32))
    for c in range(n_chunks):
        h_ref[c * rc:(c + 1) * rc, :] = hs[c]

    @pl.when(ti == pl.num_programs(0) - 1)
    def _():
        h = jnp.concatenate(hs, axis=0)
        z1 = (jnp.dot(h, w1_ref[...], preferred_element_type=jnp.float32)
              + b1_ref[...])
        z1 = jnp.maximum(z1, 0.0)
        zz = z1 * w2_ref[...]
        out = (jnp.dot(zz, s_ref[...], preferred_element_type=jnp.float32)
               + b2_ref[0, 0])
        out_ref[...] = out


def _pick_t_tile(T, cap=32):
    best = 1
    for cand in range(1, min(T, cap) + 1):
        if T % cand == 0:
            best = cand
    return best


@jax.jit
def kernel(x, w_ih, w_hh, b_ih, b_hh, w_fc1, b_fc1, w_fc2, b_fc2):
    B, T, I = x.shape
    H = w_hh.shape[0]
    F = w_fc1.shape[1]

    G = 4
    assert B % G == 0
    R = B // G
    assert R % 8 == 0
    t_tile = _pick_t_tile(T)
    nt = T // t_tile
    n_chunks = 2 if R % 16 == 0 else 1

    f32 = jnp.float32
    bf16 = jnp.bfloat16
    x = x.astype(bf16)

    x_rows = (x.reshape(G, R, T, I)
              .transpose(2, 1, 0, 3)
              .reshape(T * R, G * I))

    eye = jnp.eye(G, dtype=f32)
    wih_bd = jnp.kron(eye, w_ih.astype(f32)).astype(bf16)
    whh_bd = jnp.kron(eye, w_hh.astype(f32))
    b_pk = jnp.tile((b_ih + b_hh).astype(f32), (1, G))
    w1_bd = jnp.kron(eye, w_fc1.astype(f32))
    b1_pk = jnp.tile(b_fc1.astype(f32), (1, G))
    w2_pk = jnp.tile(w_fc2.astype(f32).reshape(1, F), (1, G))
    seg = jnp.kron(eye, jnp.ones((F, 1), f32))
    b2 = jnp.asarray(b_fc2, f32).reshape(1, 1)

    cost = pl.CostEstimate(
        flops=2 * T * B * (I * H + H * H) + 2 * B * (H * F + F),
        transcendentals=T * B * H,
        bytes_accessed=4 * (T * B * I + B),
    )

    grid_spec = pltpu.PrefetchScalarGridSpec(
        num_scalar_prefetch=0,
        grid=(nt,),
        in_specs=[
            pl.BlockSpec((t_tile * R, G * I), lambda t: (t, 0)),
            pl.BlockSpec((G * I, G * H), lambda t: (0, 0)),
            pl.BlockSpec((G * H, G * H), lambda t: (0, 0)),
            pl.BlockSpec((1, G * H), lambda t: (0, 0)),
            pl.BlockSpec((G * H, G * F), lambda t: (0, 0)),
            pl.BlockSpec((1, G * F), lambda t: (0, 0)),
            pl.BlockSpec((1, G * F), lambda t: (0, 0)),
            pl.BlockSpec((G * F, G), lambda t: (0, 0)),
            pl.BlockSpec((1, 1), lambda t: (0, 0)),
        ],
        out_specs=pl.BlockSpec((R, G), lambda t: (0, 0)),
        scratch_shapes=[pltpu.VMEM((R, G * H), f32)],
    )

    out = pl.pallas_call(
        functools.partial(_rnn_packed_kernel, t_tile=t_tile,
                          n_chunks=n_chunks),
        out_shape=jax.ShapeDtypeStruct((R, G), f32),
        grid_spec=grid_spec,
        compiler_params=pltpu.CompilerParams(
            dimension_semantics=("arbitrary",),
            vmem_limit_bytes=48 * 1024 * 1024,
        ),
        cost_estimate=cost,
    )(x_rows, wih_bd, whh_bd, b_pk, w1_bd, b1_pk, w2_pk, seg, b2)

    return out.transpose(1, 0).reshape(B)
